# Initial kernel scaffold; baseline (speedup 1.0000x reference)
import jax
import jax.numpy as jnp
from jax.experimental import pallas as pl


def kernel(x, Wqkv, bqkv, Wproj, bproj):
    raise NotImplementedError("write your pallas kernel here")


# speedup vs baseline: 17.8049x; 17.8049x over previous
import functools
import jax
import jax.numpy as jnp
from jax import lax
from jax.experimental import pallas as pl
from jax.experimental.pallas import tpu as pltpu

NEG_INF = float("-inf")


def _qkv_kernel(x_ref, w_ref, b_ref, o_ref):
    acc = lax.dot_general(
        x_ref[...], w_ref[...],
        dimension_numbers=(((1,), (1,)), ((), ())),
        preferred_element_type=jnp.float32,
        precision=lax.Precision.DEFAULT,
    )
    o_ref[0] = acc + b_ref[0]


def _attn_kernel(k_top, scale, q_ref, k_ref, v_ref, o_ref, s_ref):
    q = q_ref[0]
    k = k_ref[0]
    s = lax.dot_general(
        q, k,
        dimension_numbers=(((1,), (1,)), ((), ())),
        preferred_element_type=jnp.float32,
        precision=lax.Precision.DEFAULT,
    ) * scale
    s_ref[...] = s
    m = jnp.max(s, axis=1, keepdims=True)

    def body(_, t):
        return jnp.max(jnp.where(s_ref[...] < t, s_ref[...], NEG_INF),
                       axis=1, keepdims=True)

    t = lax.fori_loop(0, k_top - 1, body, m)
    s = s_ref[...]
    p = jnp.where(s >= t, jnp.exp(s - m), 0.0)
    denom = jnp.sum(p, axis=1, keepdims=True)
    o = lax.dot_general(
        p, v_ref[0],
        dimension_numbers=(((1,), (0,)), ((), ())),
        preferred_element_type=jnp.float32,
        precision=lax.Precision.DEFAULT,
    )
    o_ref[0] = o / denom


def _proj_kernel(x_ref, w_ref, b_ref, o_ref):
    acc = lax.dot_general(
        x_ref[...], w_ref[...],
        dimension_numbers=(((1,), (1,)), ((), ())),
        preferred_element_type=jnp.float32,
        precision=lax.Precision.DEFAULT,
    )
    o_ref[...] = acc + b_ref[0]


@jax.jit
def kernel(x, Wqkv, bqkv, Wproj, bproj):
    B, N, C = x.shape
    H = 16
    HD = C // H
    K_TOP = 20
    scale = HD ** -0.5
    x2 = x.reshape(N, C)

    nslab = 3 * H
    qkv = pl.pallas_call(
        _qkv_kernel,
        grid=(nslab,),
        in_specs=[
            pl.BlockSpec((N, C), lambda j: (0, 0)),
            pl.BlockSpec((HD, C), lambda j: (j, 0)),
            pl.BlockSpec((1, 1, HD), lambda j: (j, 0, 0)),
        ],
        out_specs=pl.BlockSpec((1, N, HD), lambda j: (j, 0, 0)),
        out_shape=jax.ShapeDtypeStruct((nslab, N, HD), jnp.float32),
    )(x2, Wqkv, bqkv.reshape(nslab, 1, HD))

    attn_out = pl.pallas_call(
        functools.partial(_attn_kernel, K_TOP, scale),
        grid=(H,),
        in_specs=[
            pl.BlockSpec((1, N, HD), lambda h: (h, 0, 0)),
            pl.BlockSpec((1, N, HD), lambda h: (h + H, 0, 0)),
            pl.BlockSpec((1, N, HD), lambda h: (h + 2 * H, 0, 0)),
        ],
        out_specs=pl.BlockSpec((1, N, HD), lambda h: (h, 0, 0)),
        out_shape=jax.ShapeDtypeStruct((H, N, HD), jnp.float32),
        scratch_shapes=[pltpu.VMEM((N, N), jnp.float32)],
    )(qkv, qkv, qkv)

    attn_flat = attn_out.transpose(1, 0, 2).reshape(N, C)

    out = pl.pallas_call(
        _proj_kernel,
        grid=(H,),
        in_specs=[
            pl.BlockSpec((N, C), lambda j: (0, 0)),
            pl.BlockSpec((HD, C), lambda j: (j, 0)),
            pl.BlockSpec((1, 1, HD), lambda j: (j, 0, 0)),
        ],
        out_specs=pl.BlockSpec((N, HD), lambda j: (0, j)),
        out_shape=jax.ShapeDtypeStruct((N, C), jnp.float32),
    )(attn_flat, Wproj, bproj.reshape(H, 1, HD))

    return out.reshape(B, N, C)


# speedup vs baseline: 27.9567x; 1.5702x over previous
import functools
import jax
import jax.numpy as jnp
from jax import lax
from jax.experimental import pallas as pl
from jax.experimental.pallas import tpu as pltpu

NEG_INF = float("-inf")


def _qkv_kernel(x_ref, w_ref, b_ref, o_ref):
    acc = lax.dot_general(
        x_ref[...], w_ref[...],
        dimension_numbers=(((1,), (1,)), ((), ())),
        preferred_element_type=jnp.float32,
        precision=lax.Precision.DEFAULT,
    )
    o_ref[0] = acc + b_ref[0]


def _attn_kernel(k_top, scale, q_ref, k_ref, v_ref, o_ref, s_ref):
    q = q_ref[0]
    k = k_ref[0]
    s = lax.dot_general(
        q, k,
        dimension_numbers=(((1,), (1,)), ((), ())),
        preferred_element_type=jnp.float32,
        precision=lax.Precision.DEFAULT,
    ) * scale
    s_ref[...] = s
    n, w = s_ref.shape
    nc = w // 128

    r1 = s_ref[:, 0:128]
    neg = jnp.full((n, 128), NEG_INF, jnp.float32)
    r2, r3, r4 = neg, neg, neg
    for i in range(1, nc):
        x = s_ref[:, i * 128:(i + 1) * 128]
        t1 = jnp.maximum(r1, x)
        x = jnp.minimum(r1, x)
        r1 = t1
        t2 = jnp.maximum(r2, x)
        x = jnp.minimum(r2, x)
        r2 = t2
        t3 = jnp.maximum(r3, x)
        x = jnp.minimum(r3, x)
        r3 = t3
        r4 = jnp.maximum(r4, x)
    cand = jnp.concatenate([r1, r2, r3, r4], axis=1)
    m = jnp.max(r1, axis=1, keepdims=True)

    t = m
    for _ in range(k_top - 1):
        t = jnp.max(jnp.where(cand < t, cand, NEG_INF), axis=1, keepdims=True)

    s = s_ref[...]
    cnt = jnp.sum(jnp.where(s >= t, 1.0, 0.0), axis=1, keepdims=True)
    ok = jnp.all(cnt == jnp.float32(k_top))

    def fallback():
        def body(_, tt):
            return jnp.max(jnp.where(s_ref[...] < tt, s_ref[...], NEG_INF),
                           axis=1, keepdims=True)
        return lax.fori_loop(0, k_top - 1, body, m)

    t = lax.cond(ok, lambda: t, fallback)
    s = s_ref[...]
    p = jnp.where(s >= t, jnp.exp(s - m), 0.0)
    denom = jnp.sum(p, axis=1, keepdims=True)
    o = lax.dot_general(
        p, v_ref[0],
        dimension_numbers=(((1,), (0,)), ((), ())),
        preferred_element_type=jnp.float32,
        precision=lax.Precision.DEFAULT,
    )
    o_ref[0] = o / denom


def _proj_kernel(x_ref, w_ref, b_ref, o_ref):
    acc = lax.dot_general(
        x_ref[...], w_ref[...],
        dimension_numbers=(((1,), (1,)), ((), ())),
        preferred_element_type=jnp.float32,
        precision=lax.Precision.DEFAULT,
    )
    o_ref[...] = acc + b_ref[0]


@jax.jit
def kernel(x, Wqkv, bqkv, Wproj, bproj):
    B, N, C = x.shape
    H = 16
    HD = C // H
    K_TOP = 20
    scale = HD ** -0.5
    x2 = x.reshape(N, C)

    nslab = 3 * H
    qkv = pl.pallas_call(
        _qkv_kernel,
        grid=(nslab,),
        in_specs=[
            pl.BlockSpec((N, C), lambda j: (0, 0)),
            pl.BlockSpec((HD, C), lambda j: (j, 0)),
            pl.BlockSpec((1, 1, HD), lambda j: (j, 0, 0)),
        ],
        out_specs=pl.BlockSpec((1, N, HD), lambda j: (j, 0, 0)),
        out_shape=jax.ShapeDtypeStruct((nslab, N, HD), jnp.float32),
    )(x2, Wqkv, bqkv.reshape(nslab, 1, HD))

    attn_out = pl.pallas_call(
        functools.partial(_attn_kernel, K_TOP, scale),
        grid=(H,),
        in_specs=[
            pl.BlockSpec((1, N, HD), lambda h: (h, 0, 0)),
            pl.BlockSpec((1, N, HD), lambda h: (h + H, 0, 0)),
            pl.BlockSpec((1, N, HD), lambda h: (h + 2 * H, 0, 0)),
        ],
        out_specs=pl.BlockSpec((1, N, HD), lambda h: (h, 0, 0)),
        out_shape=jax.ShapeDtypeStruct((H, N, HD), jnp.float32),
        scratch_shapes=[pltpu.VMEM((N, N), jnp.float32)],
    )(qkv, qkv, qkv)

    attn_flat = attn_out.transpose(1, 0, 2).reshape(N, C)

    out = pl.pallas_call(
        _proj_kernel,
        grid=(H,),
        in_specs=[
            pl.BlockSpec((N, C), lambda j: (0, 0)),
            pl.BlockSpec((HD, C), lambda j: (j, 0)),
            pl.BlockSpec((1, 1, HD), lambda j: (j, 0, 0)),
        ],
        out_specs=pl.BlockSpec((N, HD), lambda j: (0, j)),
        out_shape=jax.ShapeDtypeStruct((N, C), jnp.float32),
    )(attn_flat, Wproj, bproj.reshape(H, 1, HD))

    return out.reshape(B, N, C)
